# baseline (speedup 1.0000x reference)
import functools

import jax
import jax.numpy as jnp
from jax.experimental import pallas as pl
from jax.experimental.pallas import tpu as pltpu

B, S, D, DFF, E, TOP_K = 64, 1, 768, 3072, 16, 2
N = B * S
C = 4
DC = DFF // C


def _moe_kernel(x_ref, wg_ref, bg_ref, w1_ref, b1_ref, w2_ref, b2_ref,
                alpha_ref, out_ref, w_scr):
    e = pl.program_id(0)
    c = pl.program_id(1)

    @pl.when((e == 0) & (c == 0))
    def _gate():
        x = x_ref[...]
        logits = jnp.dot(x, wg_ref[...], preferred_element_type=jnp.float32)
        logits = logits + bg_ref[0, :][None, :]
        m = jnp.max(logits, axis=-1, keepdims=True)
        ex = jnp.exp(logits - m)
        probs = ex / jnp.sum(ex, axis=-1, keepdims=True)
        ids = jax.lax.broadcasted_iota(jnp.int32, (N, E), 1)
        s1 = jnp.max(probs, axis=-1, keepdims=True)
        i1 = jnp.min(jnp.where(probs == s1, ids, E), axis=-1, keepdims=True)
        probs2 = jnp.where(ids == i1, -jnp.inf, probs)
        s2 = jnp.max(probs2, axis=-1, keepdims=True)
        i2 = jnp.min(jnp.where(probs2 == s2, ids, E), axis=-1, keepdims=True)
        w = jnp.where(ids == i1, s1, 0.0) + jnp.where(ids == i2, s2, 0.0)
        w_scr[...] = w * alpha_ref[0, :][None, :]

    x = x_ref[...]
    h = jnp.dot(x, w1_ref[0], preferred_element_type=jnp.float32)
    h = h + b1_ref[0]
    h = 0.5 * h * (1.0 + jax.lax.erf(h * 0.7071067811865476))
    o = jnp.dot(h, w2_ref[0], preferred_element_type=jnp.float32)
    ids = jax.lax.broadcasted_iota(jnp.int32, (N, E), 1)
    wcol = jnp.sum(jnp.where(ids == e, w_scr[...], 0.0), axis=-1,
                   keepdims=True)

    o = jnp.where(c == 0, o + b2_ref[0], o)
    o = o * wcol

    @pl.when((e == 0) & (c == 0))
    def _init():
        out_ref[...] = o

    @pl.when((e > 0) | (c > 0))
    def _acc():
        out_ref[...] += o


@functools.partial(jax.jit, static_argnames=("interpret",))
def _moe(x, Wg, bg2, W1, b1, W2, b2, alpha2, interpret=False):
    return pl.pallas_call(
        _moe_kernel,
        grid=(E, C),
        in_specs=[
            pl.BlockSpec((N, D), lambda e, c: (0, 0)),
            pl.BlockSpec((D, E), lambda e, c: (0, 0)),
            pl.BlockSpec((1, E), lambda e, c: (0, 0)),
            pl.BlockSpec((1, D, DC), lambda e, c: (e, 0, c)),
            pl.BlockSpec((1, 1, DC), lambda e, c: (e, 0, c)),
            pl.BlockSpec((1, DC, D), lambda e, c: (e, c, 0)),
            pl.BlockSpec((1, 1, D), lambda e, c: (e, 0, 0)),
            pl.BlockSpec((1, E), lambda e, c: (0, 0)),
        ],
        out_specs=pl.BlockSpec((N, D), lambda e, c: (0, 0)),
        out_shape=jax.ShapeDtypeStruct((N, D), jnp.float32),
        scratch_shapes=[pltpu.VMEM((N, E), jnp.float32)],
        compiler_params=pltpu.CompilerParams(
            dimension_semantics=("arbitrary", "arbitrary"),
        ),
        interpret=interpret,
    )(x, Wg, bg2, W1, b1, W2, b2, alpha2)


def kernel(hidden_states, Wg, bg, W1, b1, W2, b2, alpha):
    b, s, d = hidden_states.shape
    x = hidden_states.reshape(-1, d)
    out = _moe(x, Wg, bg.reshape(1, E), W1, b1.reshape(E, 1, DFF), W2,
               b2.reshape(E, 1, D), alpha.reshape(1, E))
    return out.reshape(b, s, d)


# speedup vs baseline: 1.1108x; 1.1108x over previous
import functools

import jax
import jax.numpy as jnp
from jax.experimental import pallas as pl
from jax.experimental.pallas import tpu as pltpu

B, S, D, DFF, E, TOP_K = 64, 1, 768, 3072, 16, 2
N = B * S
H = DFF // 2


def _moe_kernel(x_ref, wg_ref, bg_ref, w1a_ref, w1b_ref, b1_ref,
                w2a_ref, w2b_ref, b2_ref, alpha_ref, out_ref, w_scr):
    e = pl.program_id(0)

    @pl.when(e == 0)
    def _gate():
        x = x_ref[...]
        logits = jnp.dot(x, wg_ref[...], preferred_element_type=jnp.float32)
        logits = logits + bg_ref[0, :][None, :]
        m = jnp.max(logits, axis=-1, keepdims=True)
        ex = jnp.exp(logits - m)
        probs = ex / jnp.sum(ex, axis=-1, keepdims=True)
        ids = jax.lax.broadcasted_iota(jnp.int32, (N, E), 1)
        s1 = jnp.max(probs, axis=-1, keepdims=True)
        i1 = jnp.min(jnp.where(probs == s1, ids, E), axis=-1, keepdims=True)
        probs2 = jnp.where(ids == i1, -jnp.inf, probs)
        s2 = jnp.max(probs2, axis=-1, keepdims=True)
        i2 = jnp.min(jnp.where(probs2 == s2, ids, E), axis=-1, keepdims=True)
        w = jnp.where(ids == i1, s1, 0.0) + jnp.where(ids == i2, s2, 0.0)
        w_scr[...] = w * alpha_ref[0, :][None, :]

    def gelu(v):
        return 0.5 * v * (1.0 + jax.lax.erf(v * 0.7071067811865476))

    x = x_ref[...]
    ha = jnp.dot(x, w1a_ref[0], preferred_element_type=jnp.float32)
    hb = jnp.dot(x, w1b_ref[0], preferred_element_type=jnp.float32)
    ga = gelu(ha + b1_ref[0, :, :H])
    gb = gelu(hb + b1_ref[0, :, H:])
    o = jnp.dot(ga, w2a_ref[0], preferred_element_type=jnp.float32)
    o = o + jnp.dot(gb, w2b_ref[0], preferred_element_type=jnp.float32)
    o = o + b2_ref[0]
    ids = jax.lax.broadcasted_iota(jnp.int32, (N, E), 1)
    wcol = jnp.sum(jnp.where(ids == e, w_scr[...], 0.0), axis=-1,
                   keepdims=True)
    o = o * wcol

    @pl.when(e == 0)
    def _init():
        out_ref[...] = o

    @pl.when(e > 0)
    def _acc():
        out_ref[...] += o


@functools.partial(jax.jit, static_argnames=("interpret",))
def _moe(x, Wg, bg2, W1, b1, W2, b2, alpha2, interpret=False):
    return pl.pallas_call(
        _moe_kernel,
        grid=(E,),
        in_specs=[
            pl.BlockSpec((N, D), lambda e: (0, 0)),
            pl.BlockSpec((D, E), lambda e: (0, 0)),
            pl.BlockSpec((1, E), lambda e: (0, 0)),
            pl.BlockSpec((1, D, H), lambda e: (e, 0, 0)),
            pl.BlockSpec((1, D, H), lambda e: (e, 0, 1)),
            pl.BlockSpec((1, 1, DFF), lambda e: (e, 0, 0)),
            pl.BlockSpec((1, H, D), lambda e: (e, 0, 0)),
            pl.BlockSpec((1, H, D), lambda e: (e, 1, 0)),
            pl.BlockSpec((1, 1, D), lambda e: (e, 0, 0)),
            pl.BlockSpec((1, E), lambda e: (0, 0)),
        ],
        out_specs=pl.BlockSpec((N, D), lambda e: (0, 0)),
        out_shape=jax.ShapeDtypeStruct((N, D), jnp.float32),
        scratch_shapes=[pltpu.VMEM((N, E), jnp.float32)],
        compiler_params=pltpu.CompilerParams(
            dimension_semantics=("arbitrary",),
        ),
        interpret=interpret,
    )(x, Wg, bg2, W1, W1, b1, W2, W2, b2, alpha2)


def kernel(hidden_states, Wg, bg, W1, b1, W2, b2, alpha):
    b, s, d = hidden_states.shape
    x = hidden_states.reshape(-1, d)
    out = _moe(x, Wg, bg.reshape(1, E), W1, b1.reshape(E, 1, DFF), W2,
               b2.reshape(E, 1, D), alpha.reshape(1, E))
    return out.reshape(b, s, d)


# speedup vs baseline: 1.1175x; 1.0060x over previous
import functools

import jax
import jax.numpy as jnp
from jax.experimental import pallas as pl
from jax.experimental.pallas import tpu as pltpu

B, S, D, DFF, E, TOP_K = 64, 1, 768, 3072, 16, 2
N = B * S
H = DFF // 4


def _moe_kernel(x_ref, wg_ref, bg_ref, w1a_ref, w1b_ref, w1c_ref, w1d_ref,
                b1_ref, w2a_ref, w2b_ref, w2c_ref, w2d_ref, b2_ref,
                alpha_ref, out_ref, w_scr):
    e = pl.program_id(0)

    @pl.when(e == 0)
    def _gate():
        x = x_ref[...]
        logits = jnp.dot(x, wg_ref[...], preferred_element_type=jnp.float32)
        logits = logits + bg_ref[0, :][None, :]
        m = jnp.max(logits, axis=-1, keepdims=True)
        ex = jnp.exp(logits - m)
        probs = ex / jnp.sum(ex, axis=-1, keepdims=True)
        ids = jax.lax.broadcasted_iota(jnp.int32, (N, E), 1)
        s1 = jnp.max(probs, axis=-1, keepdims=True)
        i1 = jnp.min(jnp.where(probs == s1, ids, E), axis=-1, keepdims=True)
        probs2 = jnp.where(ids == i1, -jnp.inf, probs)
        s2 = jnp.max(probs2, axis=-1, keepdims=True)
        i2 = jnp.min(jnp.where(probs2 == s2, ids, E), axis=-1, keepdims=True)
        w = jnp.where(ids == i1, s1, 0.0) + jnp.where(ids == i2, s2, 0.0)
        w_scr[...] = w * alpha_ref[0, :][None, :]

    def gelu(v):
        return 0.5 * v * (1.0 + jax.lax.erf(v * 0.7071067811865476))

    x = x_ref[...]
    o = b2_ref[0]
    for q, (w1q, w2q) in enumerate(((w1a_ref, w2a_ref), (w1b_ref, w2b_ref),
                                    (w1c_ref, w2c_ref), (w1d_ref, w2d_ref))):
        hq = jnp.dot(x, w1q[0], preferred_element_type=jnp.float32)
        gq = gelu(hq + b1_ref[0, :, q * H:(q + 1) * H])
        o = o + jnp.dot(gq, w2q[0], preferred_element_type=jnp.float32)
    ids = jax.lax.broadcasted_iota(jnp.int32, (N, E), 1)
    wcol = jnp.sum(jnp.where(ids == e, w_scr[...], 0.0), axis=-1,
                   keepdims=True)
    o = o * wcol

    @pl.when(e == 0)
    def _init():
        out_ref[...] = o

    @pl.when(e > 0)
    def _acc():
        out_ref[...] += o


@functools.partial(jax.jit, static_argnames=("interpret",))
def _moe(x, Wg, bg2, W1, b1, W2, b2, alpha2, interpret=False):
    return pl.pallas_call(
        _moe_kernel,
        grid=(E,),
        in_specs=[
            pl.BlockSpec((N, D), lambda e: (0, 0)),
            pl.BlockSpec((D, E), lambda e: (0, 0)),
            pl.BlockSpec((1, E), lambda e: (0, 0)),
            pl.BlockSpec((1, D, H), lambda e: (e, 0, 0)),
            pl.BlockSpec((1, D, H), lambda e: (e, 0, 1)),
            pl.BlockSpec((1, D, H), lambda e: (e, 0, 2)),
            pl.BlockSpec((1, D, H), lambda e: (e, 0, 3)),
            pl.BlockSpec((1, 1, DFF), lambda e: (e, 0, 0)),
            pl.BlockSpec((1, H, D), lambda e: (e, 0, 0)),
            pl.BlockSpec((1, H, D), lambda e: (e, 1, 0)),
            pl.BlockSpec((1, H, D), lambda e: (e, 2, 0)),
            pl.BlockSpec((1, H, D), lambda e: (e, 3, 0)),
            pl.BlockSpec((1, 1, D), lambda e: (e, 0, 0)),
            pl.BlockSpec((1, E), lambda e: (0, 0)),
        ],
        out_specs=pl.BlockSpec((N, D), lambda e: (0, 0)),
        out_shape=jax.ShapeDtypeStruct((N, D), jnp.float32),
        scratch_shapes=[pltpu.VMEM((N, E), jnp.float32)],
        compiler_params=pltpu.CompilerParams(
            dimension_semantics=("arbitrary",),
        ),
        interpret=interpret,
    )(x, Wg, bg2, W1, W1, W1, W1, b1, W2, W2, W2, W2, b2, alpha2)


def kernel(hidden_states, Wg, bg, W1, b1, W2, b2, alpha):
    b, s, d = hidden_states.shape
    x = hidden_states.reshape(-1, d)
    out = _moe(x, Wg, bg.reshape(1, E), W1, b1.reshape(E, 1, DFF), W2,
               b2.reshape(E, 1, D), alpha.reshape(1, E))
    return out.reshape(b, s, d)
